# baseline (speedup 1.0000x reference)
import functools
import math

import jax
import jax.numpy as jnp
from jax import lax
from jax.experimental import pallas as pl
from jax.experimental.pallas import tpu as pltpu
from jax.experimental.pallas import tpu_sc as plsc

_B = 512
_T = 6
_NPTS = 2000
_NPAD = 2048
_CH2 = _NPAD // 32
_NW = 32
_BPW = _B // _NW


def _sc_body(lxy_hbm, sg_hbm, out_hbm, bufA, bufB, sgA, sgB, vtab, outb, semA, semB):
    wid = lax.axis_index("s") * 2 + lax.axis_index("c")
    iot = lax.iota(jnp.int32, 16)
    base_b = wid * _BPW

    def vinit(c, _):
        vtab[pl.ds(c * 16, 16)] = lax.div(c * 16 + iot, jnp.int32(20))
        return 0

    lax.fori_loop(0, _NPAD // 16, vinit, 0)

    def issue(b, buf, sg, sem):
        pltpu.async_copy(lxy_hbm.at[pl.ds(b * (2 * _NPTS), 2 * _NPTS)], buf, sem)
        pltpu.async_copy(sg_hbm.at[b], sg, sem)

    def wait(b, buf, sg, sem):
        pltpu.make_async_copy(
            lxy_hbm.at[pl.ds(b * (2 * _NPTS), 2 * _NPTS)], buf, sem).wait()
        pltpu.make_async_copy(sg_hbm.at[b], sg, sem).wait()

    def process(slot, buf, sg):
        ev = sg[pl.ds(112, 16)]
        m2p = []
        m2q = []
        ax_ = ev[0] * 0.0
        ay_ = ax_
        for t in range(_T):
            ax_ = ax_ + ev[2 * t]
            ay_ = ay_ + ev[2 * t + 1]
            m2p.append(-2.0 * ax_)
            m2q.append(-2.0 * ay_)

        big = jnp.full((16,), 3e38, jnp.float32)
        zi = jnp.zeros((16,), jnp.int32)
        init = (tuple(big for _ in range(_T)), tuple(zi for _ in range(_T)))
        iot2 = iot * 2

        def cbody(c, carry):
            mins, idxs = carry
            for k in range(2):
                base = c * 32 + k * 16
                off = jnp.minimum(base * 2 + iot2, jnp.int32(3998))
                xr = plsc.load_gather(buf, [off])
                yr = plsc.load_gather(buf, [off + 1])
                vix = vtab[pl.ds(base, 16)]
                scv = plsc.load_gather(sg, [vix])
                m = scv < 0.5
                xc = xr * 30.0 - 15.0
                yc = yr * 60.0 - 30.0
                r2 = xc * xc + yc * yc
                r2 = jnp.where(m, 1e30, r2)
                fidx = base + iot
                nm = []
                ni = []
                for t in range(_T):
                    e = xc * m2p[t] + r2
                    e = yc * m2q[t] + e
                    lt = e < mins[t]
                    nm.append(jnp.where(lt, e, mins[t]))
                    ni.append(jnp.where(lt, fidx, idxs[t]))
                mins = tuple(nm)
                idxs = tuple(ni)
            return mins, idxs

        mins, idxs = lax.fori_loop(0, _CH2, cbody, init)

        idxv = jnp.zeros((16,), jnp.int32)
        for t in range(_T):
            gmin = jnp.min(mins[t])
            ii = jnp.where(mins[t] == gmin, idxs[t], jnp.int32(2147483647))
            gidx = jnp.min(ii)
            pstar = lax.rem(gidx, jnp.int32(20))
            gnext = jnp.where(pstar == jnp.int32(19), gidx - 1, gidx + 1)
            idxv = jnp.where(iot == t, gidx, idxv)
            idxv = jnp.where(iot == t + 8, gnext, idxv)

        gxr = plsc.load_gather(buf, [idxv * 2])
        gyr = plsc.load_gather(buf, [idxv * 2 + 1])
        vig = plsc.load_gather(vtab, [idxv])
        scg = plsc.load_gather(sg, [vig])
        mg = scg < 0.5
        gx = jnp.where(mg, 1e6, gxr * 30.0 - 15.0)
        gy = jnp.where(mg, 1e6, gyr * 60.0 - 30.0)
        outb[pl.ds(slot * 32, 16)] = gx
        outb[pl.ds(slot * 32 + 16, 16)] = gy

    issue(base_b, bufA, sgA, semA)

    def bbody(j, _):
        b0 = base_b + 2 * j
        issue(b0 + 1, bufB, sgB, semB)
        wait(b0, bufA, sgA, semA)
        process(2 * j, bufA, sgA)

        @pl.when(j < _BPW // 2 - 1)
        def _():
            issue(b0 + 2, bufA, sgA, semA)

        wait(b0 + 1, bufB, sgB, semB)
        process(2 * j + 1, bufB, sgB)
        return 0

    lax.fori_loop(0, _BPW // 2, bbody, 0)
    pltpu.sync_copy(outb, out_hbm.at[pl.ds(wid * (_BPW * 32), _BPW * 32)])


_sc_kernel = functools.partial(
    pl.kernel,
    out_type=jax.ShapeDtypeStruct((_B * 32,), jnp.float32),
    mesh=plsc.VectorSubcoreMesh(
        core_axis_name="c", subcore_axis_name="s", num_cores=2, num_subcores=16
    ),
    scratch_types=[
        pltpu.VMEM((2 * _NPTS,), jnp.float32),
        pltpu.VMEM((2 * _NPTS,), jnp.float32),
        pltpu.VMEM((128,), jnp.float32),
        pltpu.VMEM((128,), jnp.float32),
        pltpu.VMEM((_NPAD,), jnp.int32),
        pltpu.VMEM((_BPW * 32,), jnp.float32),
        pltpu.SemaphoreType.DMA,
        pltpu.SemaphoreType.DMA,
    ],
    compiler_params=pltpu.CompilerParams(needs_layout_passes=False),
)(_sc_body)


def _tc_body(ex_ref, ey_ref, sc_ref, o_ref):
    exv = ex_ref[...]
    eyv = ey_ref[...]
    s = sc_ref[...]

    pxs = [exv[:, 0:1]]
    pys = [eyv[:, 0:1]]
    for t in range(1, _T):
        pxs.append(pxs[-1] + exv[:, t:t + 1])
        pys.append(pys[-1] + eyv[:, t:t + 1])
    pxc = jnp.concatenate(pxs, axis=1)
    pyc = jnp.concatenate(pys, axis=1)

    mx = s[:, 0:6]
    nx = s[:, 8:14]
    my = s[:, 16:22]
    ny = s[:, 24:30]
    bx = nx - mx
    by = ny - my

    ax = jnp.concatenate([exv[:, 1:6], exv[:, 5:6]], axis=1)
    ay = jnp.concatenate([eyv[:, 1:6], eyv[:, 5:6]], axis=1)

    cross = ax * by - ay * bx
    dot = ax * bx + ay * by
    ac = jnp.abs(cross)
    ad = jnp.abs(dot)
    mn = jnp.minimum(ac, ad)
    mxv = jnp.maximum(ac, ad)
    q = mn / (mxv + 1e-30)
    s2 = q * q
    at = ((((-0.013887473 * s2 + 0.058559403) * s2 - 0.122270391) * s2
           + 0.196054836) * s2 - 0.333060156) * s2 + 0.999998017
    at = at * q
    yaw = jnp.where(ac <= ad, at, (math.pi / 2) - at)

    ddx = mx - pxc
    ddy = my - pyc
    dmask = (ddx * ddx + ddy * ddy) > 4.0
    sdx = pxc[:, 5:6] - pxc[:, 0:1]
    sdy = pyc[:, 5:6] - pyc[:, 0:1]
    smask = (sdx * sdx + sdy * sdy) < 1.0
    yaw = jnp.where(dmask | smask, 0.0, yaw)
    o_ref[...] = jnp.sum(yaw).reshape(1, 1) * (1.0 / (_B * _T))


_tc_call = pl.pallas_call(
    _tc_body,
    out_shape=jax.ShapeDtypeStruct((1, 1), jnp.float32),
)


def kernel(ego_fut_preds, lane_preds, lane_score_preds):
    lxy = lane_preds.reshape(_B * 2 * _NPTS)
    eg = ego_fut_preds.reshape(_B, 12)
    z12 = jnp.zeros((_B, 12), jnp.float32)
    z4 = jnp.zeros((_B, 4), jnp.float32)
    sg = jnp.concatenate([lane_score_preds[:, :, 0], z12, eg, z4], axis=1)
    scout = _sc_kernel(lxy, sg)
    ex = jnp.pad(ego_fut_preds[:, :, 0], ((0, 0), (0, 2)))
    ey = jnp.pad(ego_fut_preds[:, :, 1], ((0, 0), (0, 2)))
    out = _tc_call(ex, ey, scout.reshape(_B, 32))
    return out[0, 0]


# speedup vs baseline: 18.7820x; 18.7820x over previous
import functools
import math

import jax
import jax.numpy as jnp
from jax import lax
from jax.experimental import pallas as pl
from jax.experimental.pallas import tpu as pltpu
from jax.experimental.pallas import tpu_sc as plsc

_B = 512
_T = 6
_NPTS = 2000
_NPAD = 2048
_CH2 = _NPAD // 32
_NW = 32
_BPW = _B // _NW


def _sc_body(lxy_hbm, sg_hbm, out_hbm, bufA, bufB, sgA, sgB, vtab, outb,
             semA, semB):
    wid = lax.axis_index("s") * 2 + lax.axis_index("c")
    iot = lax.iota(jnp.int32, 16)
    base_b = wid * _BPW

    def vinit(c, _):
        fidx = c * 16 + iot
        v = jnp.minimum(lax.div(fidx, jnp.int32(20)), jnp.int32(99))
        vtab[pl.ds(c * 16, 16)] = v
        return 0

    lax.fori_loop(0, _NPAD // 16, vinit, 0)
    zc = jnp.zeros((16,), jnp.int32)
    oc = jnp.ones((16,), jnp.int32)

    def issue(b, buf, sg, sem):
        pltpu.async_copy(lxy_hbm.at[b], buf, sem)
        pltpu.async_copy(sg_hbm.at[b], sg, sem)

    def wait(b, buf, sg, sem):
        pltpu.make_async_copy(lxy_hbm.at[b], buf, sem).wait()
        pltpu.make_async_copy(sg_hbm.at[b], sg, sem).wait()

    def process(slot, buf, sg):
        ev = sg[pl.ds(112, 16)]
        m2p = []
        m2q = []
        ax_ = ev[0] * 0.0
        ay_ = ax_
        for t in range(_T):
            ax_ = ax_ + ev[2 * t]
            ay_ = ay_ + ev[2 * t + 1]
            m2p.append(-2.0 * ax_)
            m2q.append(-2.0 * ay_)

        big = jnp.full((16,), 3e38, jnp.float32)
        zi = jnp.zeros((16,), jnp.int32)
        init = (tuple(big for _ in range(_T)), tuple(zi for _ in range(_T)))
        iot2 = iot * 2

        def cbody(c, carry):
            mins, idxs = carry
            for k in range(2):
                base = c * 32 + k * 16
                off = jnp.minimum(base * 2 + iot2, jnp.int32(2 * _NPTS - 2))
                xr = plsc.load_gather(buf, [off])
                yr = plsc.load_gather(buf, [off + 1])
                vix = vtab[pl.ds(base, 16)]
                scv = plsc.load_gather(sg, [vix])
                m = scv < 0.5
                xc = xr * 30.0 - 15.0
                yc = yr * 60.0 - 30.0
                r2 = xc * xc + yc * yc
                r2 = jnp.where(m, 1e30, r2)
                fidx = base + iot
                nm = []
                ni = []
                for t in range(_T):
                    e = xc * m2p[t] + r2
                    e = yc * m2q[t] + e
                    lt = e < mins[t]
                    nm.append(jnp.where(lt, e, mins[t]))
                    ni.append(jnp.where(lt, fidx, idxs[t]))
                mins = tuple(nm)
                idxs = tuple(ni)
            return mins, idxs

        mins, idxs = lax.fori_loop(0, _CH2, cbody, init)

        idxv = jnp.zeros((16,), jnp.int32)
        for t in range(_T):
            gmin = jnp.min(mins[t])
            ii = jnp.where(mins[t] == gmin, idxs[t], jnp.int32(2147483647))
            gidx = jnp.min(ii)
            pstar = lax.rem(gidx, jnp.int32(20))
            gnext = jnp.where(pstar == jnp.int32(19), gidx - 1, gidx + 1)
            idxv = jnp.where(iot == t, gidx, idxv)
            idxv = jnp.where(iot == t + 8, gnext, idxv)

        gxr = plsc.load_gather(buf, [idxv * 2])
        gyr = plsc.load_gather(buf, [idxv * 2 + 1])
        vig = plsc.load_gather(vtab, [idxv])
        scg = plsc.load_gather(sg, [vig])
        mg = scg < 0.5
        gx = jnp.where(mg, 1e6, gxr * 30.0 - 15.0)
        gy = jnp.where(mg, 1e6, gyr * 60.0 - 30.0)
        outb[pl.ds(slot * 32, 16)] = gx
        outb[pl.ds(slot * 32 + 16, 16)] = gy

    issue(base_b, bufA, sgA, semA)

    def bbody(j, _):
        b0 = base_b + 2 * j
        issue(b0 + 1, bufB, sgB, semB)
        wait(b0, bufA, sgA, semA)
        process(2 * j, bufA, sgA)

        @pl.when(j < _BPW // 2 - 1)
        def _():
            issue(b0 + 2, bufA, sgA, semA)

        wait(b0 + 1, bufB, sgB, semB)
        process(2 * j + 1, bufB, sgB)
        return 0

    lax.fori_loop(0, _BPW // 2, bbody, 0)
    pltpu.sync_copy(outb, out_hbm.at[pl.ds(wid * (_BPW * 32), _BPW * 32)])


_sc_kernel = functools.partial(
    pl.kernel,
    out_type=jax.ShapeDtypeStruct((_B * 32,), jnp.float32),
    mesh=plsc.VectorSubcoreMesh(
        core_axis_name="c", subcore_axis_name="s", num_cores=2, num_subcores=16
    ),
    scratch_types=[
        pltpu.VMEM((2 * _NPTS,), jnp.float32),
        pltpu.VMEM((2 * _NPTS,), jnp.float32),
        pltpu.VMEM((128,), jnp.float32),
        pltpu.VMEM((128,), jnp.float32),
        pltpu.VMEM((_NPAD,), jnp.int32),
        pltpu.VMEM((_BPW * 32,), jnp.float32),
        pltpu.SemaphoreType.DMA,
        pltpu.SemaphoreType.DMA,
    ],
    compiler_params=pltpu.CompilerParams(
        needs_layout_passes=False, use_tc_tiling_on_sc=True
    ),
)(_sc_body)


def _tc_body(ex_ref, ey_ref, sc_ref, o_ref):
    exv = ex_ref[...]
    eyv = ey_ref[...]
    s = sc_ref[...]

    pxs = [exv[:, 0:1]]
    pys = [eyv[:, 0:1]]
    for t in range(1, _T):
        pxs.append(pxs[-1] + exv[:, t:t + 1])
        pys.append(pys[-1] + eyv[:, t:t + 1])
    pxc = jnp.concatenate(pxs, axis=1)
    pyc = jnp.concatenate(pys, axis=1)

    mx = s[:, 0:6]
    nx = s[:, 8:14]
    my = s[:, 16:22]
    ny = s[:, 24:30]
    bx = nx - mx
    by = ny - my

    ax = jnp.concatenate([exv[:, 1:6], exv[:, 5:6]], axis=1)
    ay = jnp.concatenate([eyv[:, 1:6], eyv[:, 5:6]], axis=1)

    cross = ax * by - ay * bx
    dot = ax * bx + ay * by
    ac = jnp.abs(cross)
    ad = jnp.abs(dot)
    mn = jnp.minimum(ac, ad)
    mxv = jnp.maximum(ac, ad)
    q = mn / (mxv + 1e-30)
    s2 = q * q
    at = ((((-0.013887473 * s2 + 0.058559403) * s2 - 0.122270391) * s2
           + 0.196054836) * s2 - 0.333060156) * s2 + 0.999998017
    at = at * q
    yaw = jnp.where(ac <= ad, at, (math.pi / 2) - at)

    ddx = mx - pxc
    ddy = my - pyc
    dmask = (ddx * ddx + ddy * ddy) > 4.0
    sdx = pxc[:, 5:6] - pxc[:, 0:1]
    sdy = pyc[:, 5:6] - pyc[:, 0:1]
    smask = (sdx * sdx + sdy * sdy) < 1.0
    yaw = jnp.where(dmask | smask, 0.0, yaw)
    o_ref[...] = jnp.sum(yaw).reshape(1, 1) * (1.0 / (_B * _T))


_tc_call = pl.pallas_call(
    _tc_body,
    out_shape=jax.ShapeDtypeStruct((1, 1), jnp.float32),
)


def kernel(ego_fut_preds, lane_preds, lane_score_preds):
    lxy = lane_preds.reshape(_B, 2 * _NPTS)
    eg = ego_fut_preds.reshape(_B, 12)
    z12 = jnp.zeros((_B, 12), jnp.float32)
    z4 = jnp.zeros((_B, 4), jnp.float32)
    sg = jnp.concatenate([lane_score_preds[:, :, 0], z12, eg, z4], axis=1)
    scout = _sc_kernel(lxy, sg)
    ex = jnp.pad(ego_fut_preds[:, :, 0], ((0, 0), (0, 2)))
    ey = jnp.pad(ego_fut_preds[:, :, 1], ((0, 0), (0, 2)))
    out = _tc_call(ex, ey, scout.reshape(_B, 32))
    return out[0, 0]


# speedup vs baseline: 18.9022x; 1.0064x over previous
import functools
import math

import jax
import jax.numpy as jnp
from jax import lax
from jax.experimental import pallas as pl
from jax.experimental.pallas import tpu as pltpu
from jax.experimental.pallas import tpu_sc as plsc

_B = 512
_T = 6
_NPTS = 2000
_NPAD = 2048
_CH2 = _NPAD // 32
_NW = 32
_BPW = _B // _NW


def _sc_body(lxy_hbm, sg_hbm, out_hbm, buf0, buf1, buf2, buf3,
             sg0, sg1, sg2, sg3, vtab, otab, outb, sem0, sem1, sem2, sem3):
    wid = lax.axis_index("s") * 2 + lax.axis_index("c")
    iot = lax.iota(jnp.int32, 16)
    base_b = wid * _BPW
    bufs = (buf0, buf1, buf2, buf3)
    sgs = (sg0, sg1, sg2, sg3)
    sems = (sem0, sem1, sem2, sem3)

    def vinit(c, _):
        fidx = c * 16 + iot
        v = jnp.minimum(lax.div(fidx, jnp.int32(20)), jnp.int32(99))
        vtab[pl.ds(c * 16, 16)] = v
        otab[pl.ds(c * 16, 16)] = jnp.minimum(fidx * 2, jnp.int32(2 * _NPTS - 2))
        return 0

    lax.fori_loop(0, _NPAD // 16, vinit, 0)

    def issue(b, buf, sg, sem):
        pltpu.async_copy(lxy_hbm.at[b], buf, sem)
        pltpu.async_copy(sg_hbm.at[b], sg, sem)

    def wait(b, buf, sg, sem):
        pltpu.make_async_copy(lxy_hbm.at[b], buf, sem).wait()
        pltpu.make_async_copy(sg_hbm.at[b], sg, sem).wait()

    def process(slot, buf, sg):
        ev = sg[pl.ds(112, 16)]
        m2p = []
        m2q = []
        ax_ = ev[0] * 0.0
        ay_ = ax_
        for t in range(_T):
            ax_ = ax_ + ev[2 * t]
            ay_ = ay_ + ev[2 * t + 1]
            m2p.append(-2.0 * ax_)
            m2q.append(-2.0 * ay_)

        big = jnp.full((16,), 3e38, jnp.float32)
        zi = jnp.zeros((16,), jnp.int32)
        init = (tuple(big for _ in range(_T)), tuple(zi for _ in range(_T)))

        def cbody(c, carry):
            mins, idxs = carry
            for k in range(2):
                base = c * 32 + k * 16
                off = otab[pl.ds(base, 16)]
                xr = plsc.load_gather(buf, [off])
                yr = plsc.load_gather(buf, [off + 1])
                vix = vtab[pl.ds(base, 16)]
                scv = plsc.load_gather(sg, [vix])
                m = scv < 0.5
                xc = xr * 30.0 - 15.0
                yc = yr * 60.0 - 30.0
                r2 = xc * xc + yc * yc
                r2 = jnp.where(m, 1e30, r2)
                fidx = base + iot
                nm = []
                ni = []
                for t in range(_T):
                    e = xc * m2p[t] + r2
                    e = yc * m2q[t] + e
                    lt = e < mins[t]
                    nm.append(jnp.where(lt, e, mins[t]))
                    ni.append(jnp.where(lt, fidx, idxs[t]))
                mins = tuple(nm)
                idxs = tuple(ni)
            return mins, idxs

        mins, idxs = lax.fori_loop(0, _CH2, cbody, init)

        idxv = jnp.zeros((16,), jnp.int32)
        for t in range(_T):
            gmin = jnp.min(mins[t])
            ii = jnp.where(mins[t] == gmin, idxs[t], jnp.int32(2147483647))
            gidx = jnp.min(ii)
            pstar = lax.rem(gidx, jnp.int32(20))
            gnext = jnp.where(pstar == jnp.int32(19), gidx - 1, gidx + 1)
            idxv = jnp.where(iot == t, gidx, idxv)
            idxv = jnp.where(iot == t + 8, gnext, idxv)

        gxr = plsc.load_gather(buf, [idxv * 2])
        gyr = plsc.load_gather(buf, [idxv * 2 + 1])
        vig = plsc.load_gather(vtab, [idxv])
        scg = plsc.load_gather(sg, [vig])
        mg = scg < 0.5
        gx = jnp.where(mg, 1e6, gxr * 30.0 - 15.0)
        gy = jnp.where(mg, 1e6, gyr * 60.0 - 30.0)
        outb[pl.ds(slot * 32, 16)] = gx
        outb[pl.ds(slot * 32 + 16, 16)] = gy

    for k in range(3):
        issue(base_b + k, bufs[k], sgs[k], sems[k])

    def bbody(j, _):
        for i in range(4):
            slot = 4 * j + i
            b = base_b + slot

            @pl.when(slot + 3 < _BPW)
            def _():
                issue(base_b + slot + 3, bufs[(i + 3) % 4],
                      sgs[(i + 3) % 4], sems[(i + 3) % 4])

            wait(b, bufs[i], sgs[i], sems[i])
            process(slot, bufs[i], sgs[i])
        return 0

    lax.fori_loop(0, _BPW // 4, bbody, 0)
    pltpu.sync_copy(outb, out_hbm.at[pl.ds(wid * (_BPW * 32), _BPW * 32)])


_sc_kernel = functools.partial(
    pl.kernel,
    out_type=jax.ShapeDtypeStruct((_B * 32,), jnp.float32),
    mesh=plsc.VectorSubcoreMesh(
        core_axis_name="c", subcore_axis_name="s", num_cores=2, num_subcores=16
    ),
    scratch_types=(
        [pltpu.VMEM((2 * _NPTS,), jnp.float32)] * 4
        + [pltpu.VMEM((128,), jnp.float32)] * 4
        + [
            pltpu.VMEM((_NPAD,), jnp.int32),
            pltpu.VMEM((_NPAD,), jnp.int32),
            pltpu.VMEM((_BPW * 32,), jnp.float32),
        ]
        + [pltpu.SemaphoreType.DMA] * 4
    ),
    compiler_params=pltpu.CompilerParams(needs_layout_passes=False),
)(_sc_body)


def _tc_body(ex_ref, ey_ref, sc_ref, o_ref):
    exv = ex_ref[...]
    eyv = ey_ref[...]
    s = sc_ref[...]

    pxs = [exv[:, 0:1]]
    pys = [eyv[:, 0:1]]
    for t in range(1, _T):
        pxs.append(pxs[-1] + exv[:, t:t + 1])
        pys.append(pys[-1] + eyv[:, t:t + 1])
    pxc = jnp.concatenate(pxs, axis=1)
    pyc = jnp.concatenate(pys, axis=1)

    mx = s[:, 0:6]
    nx = s[:, 8:14]
    my = s[:, 16:22]
    ny = s[:, 24:30]
    bx = nx - mx
    by = ny - my

    ax = jnp.concatenate([exv[:, 1:6], exv[:, 5:6]], axis=1)
    ay = jnp.concatenate([eyv[:, 1:6], eyv[:, 5:6]], axis=1)

    cross = ax * by - ay * bx
    dot = ax * bx + ay * by
    ac = jnp.abs(cross)
    ad = jnp.abs(dot)
    mn = jnp.minimum(ac, ad)
    mxv = jnp.maximum(ac, ad)
    q = mn / (mxv + 1e-30)
    s2 = q * q
    at = ((((-0.013887473 * s2 + 0.058559403) * s2 - 0.122270391) * s2
           + 0.196054836) * s2 - 0.333060156) * s2 + 0.999998017
    at = at * q
    yaw = jnp.where(ac <= ad, at, (math.pi / 2) - at)

    ddx = mx - pxc
    ddy = my - pyc
    dmask = (ddx * ddx + ddy * ddy) > 4.0
    sdx = pxc[:, 5:6] - pxc[:, 0:1]
    sdy = pyc[:, 5:6] - pyc[:, 0:1]
    smask = (sdx * sdx + sdy * sdy) < 1.0
    yaw = jnp.where(dmask | smask, 0.0, yaw)
    o_ref[...] = jnp.sum(yaw).reshape(1, 1) * (1.0 / (_B * _T))


_tc_call = pl.pallas_call(
    _tc_body,
    out_shape=jax.ShapeDtypeStruct((1, 1), jnp.float32),
)


def kernel(ego_fut_preds, lane_preds, lane_score_preds):
    lxy = lane_preds.reshape(_B, 2 * _NPTS)
    eg = ego_fut_preds.reshape(_B, 12)
    z12 = jnp.zeros((_B, 12), jnp.float32)
    z4 = jnp.zeros((_B, 4), jnp.float32)
    sg = jnp.concatenate([lane_score_preds[:, :, 0], z12, eg, z4], axis=1)
    scout = _sc_kernel(lxy, sg)
    ex = jnp.pad(ego_fut_preds[:, :, 0], ((0, 0), (0, 2)))
    ey = jnp.pad(ego_fut_preds[:, :, 1], ((0, 0), (0, 2)))
    out = _tc_call(ex, ey, scout.reshape(_B, 32))
    return out[0, 0]
